# baseline (speedup 1.0000x reference)
import functools

import jax
import jax.numpy as jnp
from jax import lax
from jax.experimental import pallas as pl
from jax.experimental.pallas import tpu as pltpu
from jax.experimental.pallas import tpu_sc as plsc

B = 4096
F = 26
V = 100000
D = 32
NUM = 13

NC = 2
NS = 16
NW = NC * NS

B_PER_W = B // NW


@functools.lru_cache(maxsize=None)
def _make_sc_gather():
    mesh = plsc.VectorSubcoreMesh(core_axis_name="c", subcore_axis_name="s")

    @functools.partial(
        pl.kernel,
        mesh=mesh,
        out_type=jax.ShapeDtypeStruct((B, F * D), jnp.float32),
        scratch_types=[
            pltpu.VMEM((F, B_PER_W), jnp.int32),
            pltpu.VMEM((F * B_PER_W, D), jnp.float32),
            pltpu.SemaphoreType.DMA,
            pltpu.SemaphoreType.DMA,
        ],
        compiler_params=pltpu.CompilerParams(use_tc_tiling_on_sc=False),
    )
    def _sc_gather(table_hbm, cat_hbm, out_hbm, idx_v, rows_v, gsem, wsem):
        w = lax.axis_index("s") * NC + lax.axis_index("c")
        base = w * B_PER_W
        pltpu.sync_copy(cat_hbm.at[:, pl.ds(base, B_PER_W)], idx_v)
        gathers = []
        for f in range(F):
            gathers.append(
                pltpu.async_copy(
                    table_hbm.at[f].at[idx_v.at[f]],
                    rows_v.at[pl.ds(f * B_PER_W, B_PER_W)],
                    gsem,
                )
            )
        writes = []
        for f in range(F):
            gathers[f].wait()
            writes.append(
                pltpu.async_copy(
                    rows_v.at[pl.ds(f * B_PER_W, B_PER_W)],
                    out_hbm.at[pl.ds(base, B_PER_W), pl.ds(f * D, D)],
                    wsem,
                )
            )
        for wr in writes:
            wr.wait()

    return _sc_gather


BB = 512


def _mlp_body(emb_ref, num_ref, w1a_ref, w1b_ref, b1_ref, w2_ref, b2_ref,
              w3_ref, b3_ref, w4_ref, b4_ref, out_ref):
    h = jnp.dot(emb_ref[...], w1a_ref[...], preferred_element_type=jnp.float32)
    h += jnp.dot(num_ref[...], w1b_ref[...], preferred_element_type=jnp.float32)
    h = jnp.maximum(h + b1_ref[...], 0.0)
    h = jnp.dot(h, w2_ref[...], preferred_element_type=jnp.float32)
    h = jnp.maximum(h + b2_ref[...], 0.0)
    h = jnp.dot(h, w3_ref[...], preferred_element_type=jnp.float32)
    h = jnp.maximum(h + b3_ref[...], 0.0)
    out_ref[...] = (
        jnp.dot(h, w4_ref[...], preferred_element_type=jnp.float32) + b4_ref[...]
    )


def _mlp(emb, num, w1a, w1b, b1, w2, b2, w3, b3, w4, b4):
    grid = B // BB
    full = lambda i: (0, 0)
    return pl.pallas_call(
        _mlp_body,
        grid=(grid,),
        in_specs=[
            pl.BlockSpec((BB, F * D), lambda i: (i, 0)),
            pl.BlockSpec((BB, NUM), lambda i: (i, 0)),
            pl.BlockSpec((F * D, 512), full),
            pl.BlockSpec((NUM, 512), full),
            pl.BlockSpec((1, 512), full),
            pl.BlockSpec((512, 256), full),
            pl.BlockSpec((1, 256), full),
            pl.BlockSpec((256, 128), full),
            pl.BlockSpec((1, 128), full),
            pl.BlockSpec((128, 1), full),
            pl.BlockSpec((1, 1), full),
        ],
        out_specs=pl.BlockSpec((BB, 1), lambda i: (i, 0)),
        out_shape=jax.ShapeDtypeStruct((B, 1), jnp.float32),
    )(emb, num, w1a, w1b, b1, w2, b2, w3, b3, w4, b4)


def kernel(numerical_features, cat_features, tables, W1, b1, W2, b2, W3, b3,
           W4, b4):
    emb2 = _make_sc_gather()(tables, cat_features.reshape(F, B))
    return _mlp(
        emb2,
        numerical_features,
        W1[: F * D],
        W1[F * D :],
        b1.reshape(1, -1),
        W2,
        b2.reshape(1, -1),
        W3,
        b3.reshape(1, -1),
        W4,
        b4.reshape(1, 1),
    )


# speedup vs baseline: 2.4609x; 2.4609x over previous
import functools

import jax
import jax.numpy as jnp
from jax import lax
from jax.experimental import pallas as pl
from jax.experimental.pallas import tpu as pltpu
from jax.experimental.pallas import tpu_sc as plsc

B = 4096
F = 26
V = 100000
D = 32
NUM = 13

NC = 2
NS = 16
NW = NC * NS

WLEN = 1024
NWIN_FULL = V // WLEN
TAIL = 640
VA = NWIN_FULL * WLEN + TAIL
CLEN = 8 * WLEN
NCOARSE = (V + CLEN - 1) // CLEN

NFLUSH = B // 128


def _iota16():
    return lax.iota(jnp.int32, 16)


@functools.lru_cache(maxsize=None)
def _make_k1():
    mesh = plsc.VectorSubcoreMesh(core_axis_name="c", subcore_axis_name="s")

    @functools.partial(
        pl.kernel,
        mesh=mesh,
        out_type=(
            jax.ShapeDtypeStruct((F * B // 4, 128), jnp.float32),
            jax.ShapeDtypeStruct((F, NFLUSH, 128), jnp.int32),
        ),
        scratch_types=[
            pltpu.VMEM((1, B), jnp.int32),
            pltpu.VMEM((B + 32,), jnp.int32),
            pltpu.VMEM((B + 32,), jnp.int32),
            pltpu.VMEM((B + 32,), jnp.int32),
            pltpu.VMEM((B + 32,), jnp.int32),
            pltpu.VMEM((D, WLEN), jnp.float32),
            pltpu.VMEM((D, 128), jnp.float32),
            pltpu.VMEM((1, 128), jnp.int32),
            pltpu.SemaphoreType.DMA,
            pltpu.SemaphoreType.DMA,
        ],
        compiler_params=pltpu.CompilerParams(
            use_tc_tiling_on_sc=True, needs_layout_passes=False
        ),
    )
    def k1(tt_hbm, slab_hbm, cat_hbm, packed_hbm, bidx_hbm,
           idx_v, clist_v, cblist_v, vlist_v, blist_v, blk, pack, bidx,
           wsem, fsem):
        w = lax.axis_index("s") * NC + lax.axis_index("c")

        @pl.when(w < F)
        def _body():
            pltpu.sync_copy(cat_hbm.at[w], idx_v)

            def scan(src_ref, bsrc_ref, limit, dst_ref, bdst_ref, lo, hi, first):
                def chunk(c, cnt):
                    lanes = c * 16 + _iota16()
                    if first:
                        v = idx_v[0, pl.ds(c * 16, 16)]
                        b = lanes
                    else:
                        v = src_ref[pl.ds(c * 16, 16)]
                        b = bsrc_ref[pl.ds(c * 16, 16)]
                    m = (v >= lo) & (v < hi) & (lanes < limit)
                    plsc.store_compressed(dst_ref.at[pl.ds(cnt, 16)], v, mask=m)
                    plsc.store_compressed(bdst_ref.at[pl.ds(cnt, 16)], b, mask=m)
                    npos = plsc.all_reduce_population_count(m)
                    return cnt + lax.reduce_max(npos, (0,))
                n16 = (limit + 15) >> 4 if not isinstance(limit, int) else (limit + 15) // 16
                return lax.fori_loop(0, n16, chunk, jnp.int32(0))

            def process_window(v0, wl, hi, src, cnt_c, kp0):
                cp = pltpu.async_copy(src, blk.at[:, pl.ds(0, wl)], wsem)
                wcnt = scan(clist_v, cblist_v, cnt_c,
                            vlist_v, blist_v, v0, hi, False)
                cp.wait()

                def match(j, kp):
                    vj = vlist_v[pl.ds(j, 16)][0]
                    bj = blist_v[pl.ds(j, 16)][0]
                    col = vj - v0
                    slot = kp & 127
                    row = slot >> 2
                    cb = (slot & 3) * 32
                    colv = jnp.full((16,), col, jnp.int32)
                    for h in (0, 1):
                        vals = plsc.load_gather(blk, [_iota16() + 16 * h, colv])
                        pack[row, pl.ds(cb + 16 * h, 16)] = vals
                    plsc.store_scatter(
                        bidx,
                        [jnp.zeros((16,), jnp.int32),
                         jnp.full((16,), slot, jnp.int32)],
                        jnp.full((16,), bj * F + w, jnp.int32),
                        mask=_iota16() == 0)
                    kp1 = kp + 1

                    @pl.when((kp1 & 127) == 0)
                    def _flush():
                        q = (kp1 >> 7) - 1
                        pltpu.async_copy(
                            pack, packed_hbm.at[pl.ds(w * (B // 4) + q * 32, 32)],
                            fsem).wait()
                        pltpu.async_copy(
                            bidx, bidx_hbm.at[w, pl.ds(q, 1)], fsem).wait()
                    return kp1
                return lax.fori_loop(0, wcnt, match, kp0)

            def coarse(cb, kp):
                c0 = cb * CLEN
                cnt_c = scan(None, None, B, clist_v, cblist_v,
                             c0, c0 + CLEN, True)
                def win_body(wi, kp_):
                    v0 = (cb * 8 + wi) * WLEN
                    src = tt_hbm.at[w, :, pl.ds(pl.multiple_of(v0, 128), WLEN)]
                    return process_window(v0, WLEN, v0 + WLEN, src, cnt_c, kp_)
                nw_full = lax.min(jnp.int32(8), NWIN_FULL - cb * 8)
                kp = lax.fori_loop(0, nw_full, win_body, kp)
                return kp, cnt_c

            kp = lax.fori_loop(
                0, NCOARSE - 1, lambda cb, kp_: coarse(cb, kp_)[0],
                jnp.int32(0))
            kp, cnt_c = coarse(NCOARSE - 1, kp)
            v0t = NWIN_FULL * WLEN
            kp = process_window(
                v0t, TAIL, v0t + TAIL,
                tt_hbm.at[w, :, pl.ds(v0t, TAIL)], cnt_c, kp)
            process_window(VA, 128, V, slab_hbm.at[w], cnt_c, kp)

    return k1


@functools.lru_cache(maxsize=None)
def _make_k2():
    mesh = plsc.VectorSubcoreMesh(core_axis_name="c", subcore_axis_name="s")

    @functools.partial(
        pl.kernel,
        mesh=mesh,
        out_type=jax.ShapeDtypeStruct((B * F, D), jnp.float32),
        scratch_types=[
            pltpu.VMEM((NFLUSH, 128), jnp.int32),
            pltpu.VMEM((B // 2, D), jnp.float32),
            pltpu.SemaphoreType.DMA,
            pltpu.SemaphoreType.DMA,
        ],
        compiler_params=pltpu.CompilerParams(use_tc_tiling_on_sc=False),
    )
    def k2(packed_hbm, bidx_hbm, out_hbm, bidx_v, rows_v, lsem, ssem):
        w = lax.axis_index("s") * NC + lax.axis_index("c")

        @pl.when(w < F)
        def _body():
            pltpu.sync_copy(bidx_hbm.at[w], bidx_v)
            for half in range(2):
                pltpu.sync_copy(
                    packed_hbm.at[pl.ds(w * B + half * (B // 2), B // 2)],
                    rows_v)
                scs = []
                for j in range(NFLUSH // 2):
                    scs.append(pltpu.async_copy(
                        rows_v.at[pl.ds(j * 128, 128)],
                        out_hbm.at[bidx_v.at[half * (NFLUSH // 2) + j]],
                        ssem))
                for s in scs:
                    s.wait()

    return k2


BB = 512


def _mlp_body(emb_ref, num_ref, w1a_ref, w1b_ref, b1_ref, w2_ref, b2_ref,
              w3_ref, b3_ref, w4_ref, b4_ref, out_ref):
    h = jnp.dot(emb_ref[...], w1a_ref[...], preferred_element_type=jnp.float32)
    h += jnp.dot(num_ref[...], w1b_ref[...], preferred_element_type=jnp.float32)
    h = jnp.maximum(h + b1_ref[...], 0.0)
    h = jnp.dot(h, w2_ref[...], preferred_element_type=jnp.float32)
    h = jnp.maximum(h + b2_ref[...], 0.0)
    h = jnp.dot(h, w3_ref[...], preferred_element_type=jnp.float32)
    h = jnp.maximum(h + b3_ref[...], 0.0)
    out_ref[...] = (
        jnp.dot(h, w4_ref[...], preferred_element_type=jnp.float32) + b4_ref[...]
    )


def _mlp(emb, num, w1a, w1b, b1, w2, b2, w3, b3, w4, b4):
    grid = B // BB
    full = lambda i: (0, 0)
    return pl.pallas_call(
        _mlp_body,
        grid=(grid,),
        in_specs=[
            pl.BlockSpec((BB, F * D), lambda i: (i, 0)),
            pl.BlockSpec((BB, NUM), lambda i: (i, 0)),
            pl.BlockSpec((F * D, 512), full),
            pl.BlockSpec((NUM, 512), full),
            pl.BlockSpec((1, 512), full),
            pl.BlockSpec((512, 256), full),
            pl.BlockSpec((1, 256), full),
            pl.BlockSpec((256, 128), full),
            pl.BlockSpec((1, 128), full),
            pl.BlockSpec((128, 1), full),
            pl.BlockSpec((1, 1), full),
        ],
        out_specs=pl.BlockSpec((BB, 1), lambda i: (i, 0)),
        out_shape=jax.ShapeDtypeStruct((B, 1), jnp.float32),
    )(emb, num, w1a, w1b, b1, w2, b2, w3, b3, w4, b4)


def kernel(numerical_features, cat_features, tables, W1, b1, W2, b2, W3, b3,
           W4, b4):
    tt = tables.transpose(0, 2, 1)
    slab = jnp.pad(tt[:, :, VA:], ((0, 0), (0, 0), (0, 128 - (V - VA))))
    cat3 = cat_features.reshape(F, 1, B)
    packed, bidx = _make_k1()(tt, slab, cat3)
    flat = packed.reshape(F * B, D)
    emb2 = _make_k2()(flat, bidx).reshape(B, F * D)
    return _mlp(
        emb2,
        numerical_features,
        W1[: F * D],
        W1[F * D :],
        b1.reshape(1, -1),
        W2,
        b2.reshape(1, -1),
        W3,
        b3.reshape(1, -1),
        W4,
        b4.reshape(1, 1),
    )


# speedup vs baseline: 3.6628x; 1.4884x over previous
import functools

import jax
import jax.numpy as jnp
from jax import lax
from jax.experimental import pallas as pl
from jax.experimental.pallas import tpu as pltpu
from jax.experimental.pallas import tpu_sc as plsc

B = 4096
F = 26
V = 100000
D = 32
NUM = 13

NC = 2
NS = 16
NW = NC * NS

WLEN = 1024
NWIN_FULL = V // WLEN
TAIL = 640
VA = NWIN_FULL * WLEN + TAIL
CLEN = 8 * WLEN
NCOARSE = (V + CLEN - 1) // CLEN

NFLUSH = B // 128


def _iota16():
    return lax.iota(jnp.int32, 16)


@functools.lru_cache(maxsize=None)
def _make_k1():
    mesh = plsc.VectorSubcoreMesh(core_axis_name="c", subcore_axis_name="s")

    @functools.partial(
        pl.kernel,
        mesh=mesh,
        out_type=(
            jax.ShapeDtypeStruct((F * B // 4, 128), jnp.float32),
            jax.ShapeDtypeStruct((F, NFLUSH, 128), jnp.int32),
        ),
        scratch_types=[
            pltpu.VMEM((1, B), jnp.int32),
            pltpu.VMEM((B + 32,), jnp.int32),
            pltpu.VMEM((B + 32,), jnp.int32),
            pltpu.VMEM((B + 32,), jnp.int32),
            pltpu.VMEM((B + 32,), jnp.int32),
            pltpu.VMEM((2, D, WLEN), jnp.float32),
            pltpu.VMEM((D, 128), jnp.float32),
            pltpu.VMEM((1, 128), jnp.int32),
            pltpu.SemaphoreType.DMA((2,)),
            pltpu.SemaphoreType.DMA,
        ],
        compiler_params=pltpu.CompilerParams(
            use_tc_tiling_on_sc=True, needs_layout_passes=False
        ),
    )
    def k1(tt_hbm, slab_hbm, cat_hbm, packed_hbm, bidx_hbm,
           idx_v, clist_v, cblist_v, vlist_v, blist_v, blk, pack, bidx,
           wsem, fsem):
        w = lax.axis_index("s") * NC + lax.axis_index("c")

        @pl.when(w < F)
        def _body():
            pltpu.sync_copy(cat_hbm.at[w], idx_v)

            def scan(src_ref, bsrc_ref, limit, dst_ref, bdst_ref, lo, hi, first):
                def chunk(c, cnt):
                    lanes = c * 16 + _iota16()
                    if first:
                        v = idx_v[0, pl.ds(c * 16, 16)]
                        b = lanes
                    else:
                        v = src_ref[pl.ds(c * 16, 16)]
                        b = bsrc_ref[pl.ds(c * 16, 16)]
                    m = (v >= lo) & (v < hi) & (lanes < limit)
                    plsc.store_compressed(dst_ref.at[pl.ds(cnt, 16)], v, mask=m)
                    plsc.store_compressed(bdst_ref.at[pl.ds(cnt, 16)], b, mask=m)
                    npos = plsc.all_reduce_population_count(m)
                    return cnt + lax.reduce_max(npos, (0,))
                n16 = (limit + 15) >> 4 if not isinstance(limit, int) else (limit + 15) // 16
                return lax.fori_loop(0, n16, chunk, jnp.int32(0))

            def process_window(v0, wl, hi, src, par, cnt_c, kp0,
                               prefetch=None):
                if prefetch is not None:
                    prefetch()
                wcnt = scan(clist_v, cblist_v, cnt_c,
                            vlist_v, blist_v, v0, hi, False)
                pltpu.make_async_copy(
                    src, blk.at[par, :, pl.ds(0, wl)], wsem.at[par]).wait()
                parv = jnp.full((16,), par, jnp.int32)

                def match(j, kp):
                    vj = vlist_v[pl.ds(j, 16)][0]
                    bj = blist_v[pl.ds(j, 16)][0]
                    col = vj - v0
                    slot = kp & 127
                    row = slot >> 2
                    cb = (slot & 3) * 32
                    colv = jnp.full((16,), col, jnp.int32)
                    for h in (0, 1):
                        vals = plsc.load_gather(
                            blk, [parv, _iota16() + 16 * h, colv])
                        pack[row, pl.ds(cb + 16 * h, 16)] = vals
                    plsc.store_scatter(
                        bidx,
                        [jnp.zeros((16,), jnp.int32),
                         jnp.full((16,), slot, jnp.int32)],
                        jnp.full((16,), bj * F + w, jnp.int32),
                        mask=_iota16() == 0)
                    kp1 = kp + 1

                    @pl.when((kp1 & 127) == 0)
                    def _flush():
                        q = (kp1 >> 7) - 1
                        pltpu.async_copy(
                            pack, packed_hbm.at[pl.ds(w * (B // 4) + q * 32, 32)],
                            fsem).wait()
                        pltpu.async_copy(
                            bidx, bidx_hbm.at[w, pl.ds(q, 1)], fsem).wait()
                    return kp1
                return lax.fori_loop(0, wcnt, match, kp0)

            def full_src(v0):
                return tt_hbm.at[w, :, pl.ds(pl.multiple_of(v0, 128), WLEN)]

            def coarse(cb, kp):
                c0 = cb * CLEN
                cnt_c = scan(None, None, B, clist_v, cblist_v,
                             c0, c0 + CLEN, True)
                def win_body(wi, kp_):
                    win = cb * 8 + wi
                    v0 = win * WLEN

                    def pf():
                        @pl.when(win + 1 <= NWIN_FULL - 1)
                        def _():
                            nxt = win + 1
                            pltpu.async_copy(full_src(nxt * WLEN),
                                             blk.at[nxt & 1],
                                             wsem.at[nxt & 1])
                    return process_window(v0, WLEN, v0 + WLEN, full_src(v0),
                                          win & 1, cnt_c, kp_, prefetch=pf)
                nw_full = lax.min(jnp.int32(8), NWIN_FULL - cb * 8)
                kp = lax.fori_loop(0, nw_full, win_body, kp)
                return kp, cnt_c

            pltpu.async_copy(full_src(0), blk.at[0], wsem.at[0])
            kp = lax.fori_loop(
                0, NCOARSE - 1, lambda cb, kp_: coarse(cb, kp_)[0],
                jnp.int32(0))
            v0t = NWIN_FULL * WLEN
            tail_src = tt_hbm.at[w, :, pl.ds(v0t, TAIL)]
            pltpu.async_copy(tail_src, blk.at[1, :, pl.ds(0, TAIL)],
                             wsem.at[1])
            kp, cnt_c = coarse(NCOARSE - 1, kp)

            def pf_slab():
                pltpu.async_copy(slab_hbm.at[w], blk.at[0, :, pl.ds(0, 128)],
                                 wsem.at[0])
            kp = process_window(v0t, TAIL, v0t + TAIL, tail_src, 1,
                                cnt_c, kp, prefetch=pf_slab)
            process_window(VA, 128, V, slab_hbm.at[w], 0, cnt_c, kp)

    return k1


@functools.lru_cache(maxsize=None)
def _make_k2():
    mesh = plsc.VectorSubcoreMesh(core_axis_name="c", subcore_axis_name="s")

    @functools.partial(
        pl.kernel,
        mesh=mesh,
        out_type=jax.ShapeDtypeStruct((B * F, D), jnp.float32),
        scratch_types=[
            pltpu.VMEM((NFLUSH, 128), jnp.int32),
            pltpu.VMEM((B // 2, D), jnp.float32),
            pltpu.SemaphoreType.DMA,
            pltpu.SemaphoreType.DMA,
        ],
        compiler_params=pltpu.CompilerParams(use_tc_tiling_on_sc=False),
    )
    def k2(packed_hbm, bidx_hbm, out_hbm, bidx_v, rows_v, lsem, ssem):
        w = lax.axis_index("s") * NC + lax.axis_index("c")

        @pl.when(w < F)
        def _body():
            pltpu.sync_copy(bidx_hbm.at[w], bidx_v)
            for half in range(2):
                pltpu.sync_copy(
                    packed_hbm.at[pl.ds(w * B + half * (B // 2), B // 2)],
                    rows_v)
                scs = []
                for j in range(NFLUSH // 2):
                    scs.append(pltpu.async_copy(
                        rows_v.at[pl.ds(j * 128, 128)],
                        out_hbm.at[bidx_v.at[half * (NFLUSH // 2) + j]],
                        ssem))
                for s in scs:
                    s.wait()

    return k2


BB = 512


def _mlp_body(emb_ref, num_ref, w1a_ref, w1b_ref, b1_ref, w2_ref, b2_ref,
              w3_ref, b3_ref, w4_ref, b4_ref, out_ref):
    h = jnp.dot(emb_ref[...], w1a_ref[...], preferred_element_type=jnp.float32)
    h += jnp.dot(num_ref[...], w1b_ref[...], preferred_element_type=jnp.float32)
    h = jnp.maximum(h + b1_ref[...], 0.0)
    h = jnp.dot(h, w2_ref[...], preferred_element_type=jnp.float32)
    h = jnp.maximum(h + b2_ref[...], 0.0)
    h = jnp.dot(h, w3_ref[...], preferred_element_type=jnp.float32)
    h = jnp.maximum(h + b3_ref[...], 0.0)
    out_ref[...] = (
        jnp.dot(h, w4_ref[...], preferred_element_type=jnp.float32) + b4_ref[...]
    )


def _mlp(emb, num, w1a, w1b, b1, w2, b2, w3, b3, w4, b4):
    grid = B // BB
    full = lambda i: (0, 0)
    return pl.pallas_call(
        _mlp_body,
        grid=(grid,),
        in_specs=[
            pl.BlockSpec((BB, F * D), lambda i: (i, 0)),
            pl.BlockSpec((BB, NUM), lambda i: (i, 0)),
            pl.BlockSpec((F * D, 512), full),
            pl.BlockSpec((NUM, 512), full),
            pl.BlockSpec((1, 512), full),
            pl.BlockSpec((512, 256), full),
            pl.BlockSpec((1, 256), full),
            pl.BlockSpec((256, 128), full),
            pl.BlockSpec((1, 128), full),
            pl.BlockSpec((128, 1), full),
            pl.BlockSpec((1, 1), full),
        ],
        out_specs=pl.BlockSpec((BB, 1), lambda i: (i, 0)),
        out_shape=jax.ShapeDtypeStruct((B, 1), jnp.float32),
    )(emb, num, w1a, w1b, b1, w2, b2, w3, b3, w4, b4)


def kernel(numerical_features, cat_features, tables, W1, b1, W2, b2, W3, b3,
           W4, b4):
    tt = tables.transpose(0, 2, 1)
    slab = jnp.pad(tt[:, :, VA:], ((0, 0), (0, 0), (0, 128 - (V - VA))))
    cat3 = cat_features.reshape(F, 1, B)
    packed, bidx = _make_k1()(tt, slab, cat3)
    flat = packed.reshape(F * B, D)
    emb2 = _make_k2()(flat, bidx).reshape(B, F * D)
    return _mlp(
        emb2,
        numerical_features,
        W1[: F * D],
        W1[F * D :],
        b1.reshape(1, -1),
        W2,
        b2.reshape(1, -1),
        W3,
        b3.reshape(1, -1),
        W4,
        b4.reshape(1, 1),
    )


# speedup vs baseline: 3.6994x; 1.0100x over previous
import functools

import jax
import jax.numpy as jnp
from jax import lax
from jax.experimental import pallas as pl
from jax.experimental.pallas import tpu as pltpu
from jax.experimental.pallas import tpu_sc as plsc

B = 4096
F = 26
V = 100000
D = 32
NUM = 13

NC = 2
NS = 16
NW = NC * NS

WLEN = 1024
NWIN_FULL = V // WLEN
TAIL = 640
VA = NWIN_FULL * WLEN + TAIL
CLEN = 8 * WLEN
NCOARSE = (V + CLEN - 1) // CLEN

NFLUSH = B // 128


def _iota16():
    return lax.iota(jnp.int32, 16)


@functools.lru_cache(maxsize=None)
def _make_k1():
    mesh = plsc.VectorSubcoreMesh(core_axis_name="c", subcore_axis_name="s")

    @functools.partial(
        pl.kernel,
        mesh=mesh,
        out_type=(
            jax.ShapeDtypeStruct((F * B // 4, 128), jnp.float32),
            jax.ShapeDtypeStruct((F, NFLUSH, 128), jnp.int32),
        ),
        scratch_types=[
            pltpu.VMEM((1, B), jnp.int32),
            pltpu.VMEM((B + 32,), jnp.int32),
            pltpu.VMEM((B + 32,), jnp.int32),
            pltpu.VMEM((B + 32,), jnp.int32),
            pltpu.VMEM((B + 32,), jnp.int32),
            pltpu.VMEM((2, D, WLEN), jnp.float32),
            pltpu.VMEM((2, D, 128), jnp.float32),
            pltpu.VMEM((2, 1, 128), jnp.int32),
            pltpu.SemaphoreType.DMA((2,)),
            pltpu.SemaphoreType.DMA((2,)),
        ],
        compiler_params=pltpu.CompilerParams(
            use_tc_tiling_on_sc=True, needs_layout_passes=False
        ),
    )
    def k1(tt_hbm, slab_hbm, cat_hbm, packed_hbm, bidx_hbm,
           idx_v, clist_v, cblist_v, vlist_v, blist_v, blk, pack, bidx,
           wsem, fsem):
        w = lax.axis_index("s") * NC + lax.axis_index("c")

        @pl.when(w < F)
        def _body():
            pltpu.sync_copy(cat_hbm.at[w], idx_v)

            def scan(src_ref, bsrc_ref, limit, dst_ref, bdst_ref, lo, hi, first):
                def chunk(c, cnt):
                    lanes = c * 16 + _iota16()
                    if first:
                        v = idx_v[0, pl.ds(c * 16, 16)]
                        b = lanes
                    else:
                        v = src_ref[pl.ds(c * 16, 16)]
                        b = bsrc_ref[pl.ds(c * 16, 16)]
                    m = (v >= lo) & (v < hi) & (lanes < limit)
                    plsc.store_compressed(dst_ref.at[pl.ds(cnt, 16)], v, mask=m)
                    plsc.store_compressed(bdst_ref.at[pl.ds(cnt, 16)], b, mask=m)
                    npos = plsc.all_reduce_population_count(m)
                    return cnt + lax.reduce_max(npos, (0,))
                n16 = (limit + 15) >> 4 if not isinstance(limit, int) else (limit + 15) // 16
                return lax.fori_loop(0, n16, chunk, jnp.int32(0))

            def process_window(v0, wl, hi, src, par, cnt_c, kp0,
                               prefetch=None):
                if prefetch is not None:
                    prefetch()
                wcnt = scan(clist_v, cblist_v, cnt_c,
                            vlist_v, blist_v, v0, hi, False)
                pltpu.make_async_copy(
                    src, blk.at[par, :, pl.ds(0, wl)], wsem.at[par]).wait()
                parv = jnp.full((16,), par, jnp.int32)

                def match(j, kp):
                    pp = (kp >> 7) & 1

                    @pl.when(((kp & 127) == 0) & (kp >= 256))
                    def _drain():
                        q_old = (kp >> 7) - 2
                        pltpu.make_async_copy(
                            pack.at[pp],
                            packed_hbm.at[pl.ds(w * (B // 4) + q_old * 32, 32)],
                            fsem.at[pp]).wait()
                        pltpu.make_async_copy(
                            bidx.at[pp], bidx_hbm.at[w, pl.ds(q_old, 1)],
                            fsem.at[pp]).wait()

                    vj = vlist_v[pl.ds(j, 16)][0]
                    bj = blist_v[pl.ds(j, 16)][0]
                    col = vj - v0
                    slot = kp & 127
                    row = slot >> 2
                    cb = (slot & 3) * 32
                    colv = jnp.full((16,), col, jnp.int32)
                    for h in (0, 1):
                        vals = plsc.load_gather(
                            blk, [parv, _iota16() + 16 * h, colv])
                        pack[pp, row, pl.ds(cb + 16 * h, 16)] = vals
                    plsc.store_scatter(
                        bidx,
                        [jnp.full((16,), pp, jnp.int32),
                         jnp.zeros((16,), jnp.int32),
                         jnp.full((16,), slot, jnp.int32)],
                        jnp.full((16,), bj * F + w, jnp.int32),
                        mask=_iota16() == 0)
                    kp1 = kp + 1

                    @pl.when((kp1 & 127) == 0)
                    def _flush():
                        q = (kp1 >> 7) - 1
                        qp = q & 1
                        pltpu.async_copy(
                            pack.at[qp],
                            packed_hbm.at[pl.ds(w * (B // 4) + q * 32, 32)],
                            fsem.at[qp])
                        pltpu.async_copy(
                            bidx.at[qp], bidx_hbm.at[w, pl.ds(q, 1)],
                            fsem.at[qp])
                    return kp1
                return lax.fori_loop(0, wcnt, match, kp0)

            def full_src(v0):
                return tt_hbm.at[w, :, pl.ds(pl.multiple_of(v0, 128), WLEN)]

            def coarse(cb, kp):
                c0 = cb * CLEN
                cnt_c = scan(None, None, B, clist_v, cblist_v,
                             c0, c0 + CLEN, True)
                def win_body(wi, kp_):
                    win = cb * 8 + wi
                    v0 = win * WLEN

                    def pf():
                        @pl.when(win + 1 <= NWIN_FULL - 1)
                        def _():
                            nxt = win + 1
                            pltpu.async_copy(full_src(nxt * WLEN),
                                             blk.at[nxt & 1],
                                             wsem.at[nxt & 1])
                    return process_window(v0, WLEN, v0 + WLEN, full_src(v0),
                                          win & 1, cnt_c, kp_, prefetch=pf)
                nw_full = lax.min(jnp.int32(8), NWIN_FULL - cb * 8)
                kp = lax.fori_loop(0, nw_full, win_body, kp)
                return kp, cnt_c

            pltpu.async_copy(full_src(0), blk.at[0], wsem.at[0])
            kp = lax.fori_loop(
                0, NCOARSE - 1, lambda cb, kp_: coarse(cb, kp_)[0],
                jnp.int32(0))
            v0t = NWIN_FULL * WLEN
            tail_src = tt_hbm.at[w, :, pl.ds(v0t, TAIL)]
            pltpu.async_copy(tail_src, blk.at[1, :, pl.ds(0, TAIL)],
                             wsem.at[1])
            kp, cnt_c = coarse(NCOARSE - 1, kp)

            def pf_slab():
                pltpu.async_copy(slab_hbm.at[w], blk.at[0, :, pl.ds(0, 128)],
                                 wsem.at[0])
            kp = process_window(v0t, TAIL, v0t + TAIL, tail_src, 1,
                                cnt_c, kp, prefetch=pf_slab)
            process_window(VA, 128, V, slab_hbm.at[w], 0, cnt_c, kp)
            for qq in (NFLUSH - 2, NFLUSH - 1):
                qp = qq & 1
                pltpu.make_async_copy(
                    pack.at[qp],
                    packed_hbm.at[pl.ds(w * (B // 4) + qq * 32, 32)],
                    fsem.at[qp]).wait()
                pltpu.make_async_copy(
                    bidx.at[qp], bidx_hbm.at[w, pl.ds(qq, 1)],
                    fsem.at[qp]).wait()

    return k1


@functools.lru_cache(maxsize=None)
def _make_k2():
    mesh = plsc.VectorSubcoreMesh(core_axis_name="c", subcore_axis_name="s")

    @functools.partial(
        pl.kernel,
        mesh=mesh,
        out_type=jax.ShapeDtypeStruct((B * F, D), jnp.float32),
        scratch_types=[
            pltpu.VMEM((NFLUSH, 128), jnp.int32),
            pltpu.VMEM((B // 2, D), jnp.float32),
            pltpu.SemaphoreType.DMA,
            pltpu.SemaphoreType.DMA,
        ],
        compiler_params=pltpu.CompilerParams(use_tc_tiling_on_sc=False),
    )
    def k2(packed_hbm, bidx_hbm, out_hbm, bidx_v, rows_v, lsem, ssem):
        w = lax.axis_index("s") * NC + lax.axis_index("c")

        @pl.when(w < F)
        def _body():
            pltpu.sync_copy(bidx_hbm.at[w], bidx_v)
            for half in range(2):
                pltpu.sync_copy(
                    packed_hbm.at[pl.ds(w * B + half * (B // 2), B // 2)],
                    rows_v)
                scs = []
                for j in range(NFLUSH // 2):
                    scs.append(pltpu.async_copy(
                        rows_v.at[pl.ds(j * 128, 128)],
                        out_hbm.at[bidx_v.at[half * (NFLUSH // 2) + j]],
                        ssem))
                for s in scs:
                    s.wait()

    return k2


BB = 512


def _mlp_body(emb_ref, num_ref, w1a_ref, w1b_ref, b1_ref, w2_ref, b2_ref,
              w3_ref, b3_ref, w4_ref, b4_ref, out_ref):
    h = jnp.dot(emb_ref[...], w1a_ref[...], preferred_element_type=jnp.float32)
    h += jnp.dot(num_ref[...], w1b_ref[...], preferred_element_type=jnp.float32)
    h = jnp.maximum(h + b1_ref[...], 0.0)
    h = jnp.dot(h, w2_ref[...], preferred_element_type=jnp.float32)
    h = jnp.maximum(h + b2_ref[...], 0.0)
    h = jnp.dot(h, w3_ref[...], preferred_element_type=jnp.float32)
    h = jnp.maximum(h + b3_ref[...], 0.0)
    out_ref[...] = (
        jnp.dot(h, w4_ref[...], preferred_element_type=jnp.float32) + b4_ref[...]
    )


def _mlp(emb, num, w1a, w1b, b1, w2, b2, w3, b3, w4, b4):
    grid = B // BB
    full = lambda i: (0, 0)
    return pl.pallas_call(
        _mlp_body,
        grid=(grid,),
        in_specs=[
            pl.BlockSpec((BB, F * D), lambda i: (i, 0)),
            pl.BlockSpec((BB, NUM), lambda i: (i, 0)),
            pl.BlockSpec((F * D, 512), full),
            pl.BlockSpec((NUM, 512), full),
            pl.BlockSpec((1, 512), full),
            pl.BlockSpec((512, 256), full),
            pl.BlockSpec((1, 256), full),
            pl.BlockSpec((256, 128), full),
            pl.BlockSpec((1, 128), full),
            pl.BlockSpec((128, 1), full),
            pl.BlockSpec((1, 1), full),
        ],
        out_specs=pl.BlockSpec((BB, 1), lambda i: (i, 0)),
        out_shape=jax.ShapeDtypeStruct((B, 1), jnp.float32),
    )(emb, num, w1a, w1b, b1, w2, b2, w3, b3, w4, b4)


def kernel(numerical_features, cat_features, tables, W1, b1, W2, b2, W3, b3,
           W4, b4):
    tt = tables.transpose(0, 2, 1)
    slab = jnp.pad(tt[:, :, VA:], ((0, 0), (0, 0), (0, 128 - (V - VA))))
    cat3 = cat_features.reshape(F, 1, B)
    packed, bidx = _make_k1()(tt, slab, cat3)
    flat = packed.reshape(F * B, D)
    emb2 = _make_k2()(flat, bidx).reshape(B, F * D)
    return _mlp(
        emb2,
        numerical_features,
        W1[: F * D],
        W1[F * D :],
        b1.reshape(1, -1),
        W2,
        b2.reshape(1, -1),
        W3,
        b3.reshape(1, -1),
        W4,
        b4.reshape(1, 1),
    )


# speedup vs baseline: 4.4762x; 1.2100x over previous
import functools

import jax
import jax.numpy as jnp
from jax import lax
from jax.experimental import pallas as pl
from jax.experimental.pallas import tpu as pltpu
from jax.experimental.pallas import tpu_sc as plsc

B = 4096
F = 26
V = 100000
D = 32
NUM = 13

NC = 2
NS = 16
NW = NC * NS

WLEN = 1024
NWIN_FULL = V // WLEN
TAIL = 640
VA = NWIN_FULL * WLEN + TAIL
CLEN = 8 * WLEN
NCOARSE = (V + CLEN - 1) // CLEN

NFLUSH = B // 128


def _iota16():
    return lax.iota(jnp.int32, 16)


@functools.lru_cache(maxsize=None)
def _make_k1():
    mesh = plsc.VectorSubcoreMesh(core_axis_name="c", subcore_axis_name="s")

    @functools.partial(
        pl.kernel,
        mesh=mesh,
        out_type=(
            jax.ShapeDtypeStruct((F * B // 4, 128), jnp.float32),
            jax.ShapeDtypeStruct((F, NFLUSH, 128), jnp.int32),
        ),
        scratch_types=[
            pltpu.VMEM((1, B), jnp.int32),
            pltpu.VMEM((B + 32,), jnp.int32),
            pltpu.VMEM((B + 32,), jnp.int32),
            pltpu.VMEM((B + 32,), jnp.int32),
            pltpu.VMEM((B + 32,), jnp.int32),
            pltpu.VMEM((2, D, WLEN), jnp.float32),
            pltpu.VMEM((2, D, 128), jnp.float32),
            pltpu.VMEM((2, 1, 128), jnp.int32),
            pltpu.SemaphoreType.DMA((2,)),
            pltpu.SemaphoreType.DMA((2,)),
        ],
        compiler_params=pltpu.CompilerParams(
            use_tc_tiling_on_sc=True, needs_layout_passes=False
        ),
    )
    def k1(tt_hbm, slab_hbm, cat_hbm, packed_hbm, bidx_hbm,
           idx_v, clist_v, cblist_v, vlist_v, blist_v, blk, pack, bidx,
           wsem, fsem):
        w = lax.axis_index("s") * NC + lax.axis_index("c")

        @pl.when(w < F)
        def _body():
            pltpu.sync_copy(cat_hbm.at[w], idx_v)

            def scan(src_ref, bsrc_ref, limit, dst_ref, bdst_ref, lo, hi, first):
                def chunk(c, cnt):
                    lanes = c * 16 + _iota16()
                    if first:
                        v = idx_v[0, pl.ds(c * 16, 16)]
                        b = lanes
                    else:
                        v = src_ref[pl.ds(c * 16, 16)]
                        b = bsrc_ref[pl.ds(c * 16, 16)]
                    m = (v >= lo) & (v < hi) & (lanes < limit)
                    plsc.store_compressed(dst_ref.at[pl.ds(cnt, 16)], v, mask=m)
                    plsc.store_compressed(bdst_ref.at[pl.ds(cnt, 16)], b, mask=m)
                    npos = plsc.all_reduce_population_count(m)
                    return cnt + lax.reduce_max(npos, (0,))
                n16 = (limit + 15) >> 4 if not isinstance(limit, int) else (limit + 15) // 16
                return lax.fori_loop(0, n16, chunk, jnp.int32(0))

            def process_window(v0, wl, hi, src, par, cnt_c, kp0,
                               prefetch=None):
                if prefetch is not None:
                    prefetch()
                wcnt = scan(clist_v, cblist_v, cnt_c,
                            vlist_v, blist_v, v0, hi, False)
                pltpu.make_async_copy(
                    src, blk.at[par, :, pl.ds(0, wl)], wsem.at[par]).wait()
                parv = jnp.full((16,), par, jnp.int32)

                def match16(g, kp):
                    base = g * 16
                    nv = lax.min(jnp.int32(16), wcnt - base)
                    kp_new = kp + nv
                    nb = (kp_new - 1) >> 7

                    @pl.when(((nb > (kp >> 7)) | ((kp & 127) == 0))
                             & (nb >= 2))
                    def _drain():
                        q_old = nb - 2
                        qp = q_old & 1
                        pltpu.make_async_copy(
                            pack.at[qp],
                            packed_hbm.at[pl.ds(w * (B // 4) + q_old * 32, 32)],
                            fsem.at[qp]).wait()
                        pltpu.make_async_copy(
                            bidx.at[qp], bidx_hbm.at[w, pl.ds(q_old, 1)],
                            fsem.at[qp]).wait()

                    vv = vlist_v[pl.ds(base, 16)]
                    bb = blist_v[pl.ds(base, 16)]
                    valid = _iota16() < nv
                    cols = vv - v0
                    pos = kp + _iota16()
                    ppv = (pos >> 7) & 1
                    bp32 = (pos & 127) * 32
                    for d in range(D):
                        vals = plsc.load_gather(
                            blk, [parv, jnp.full((16,), d, jnp.int32), cols],
                            mask=valid)
                        fl = bp32 + d
                        plsc.store_scatter(
                            pack, [ppv, fl >> 7, fl & 127], vals, mask=valid)
                    plsc.store_scatter(
                        bidx,
                        [ppv, jnp.zeros((16,), jnp.int32), pos & 127],
                        bb * F + w, mask=valid)

                    @pl.when((kp_new >> 7) > (kp >> 7))
                    def _flush():
                        q = (kp_new >> 7) - 1
                        qp = q & 1
                        pltpu.async_copy(
                            pack.at[qp],
                            packed_hbm.at[pl.ds(w * (B // 4) + q * 32, 32)],
                            fsem.at[qp])
                        pltpu.async_copy(
                            bidx.at[qp], bidx_hbm.at[w, pl.ds(q, 1)],
                            fsem.at[qp])
                    return kp_new
                ngrp = (wcnt + 15) >> 4
                return lax.fori_loop(0, ngrp, match16, kp0)

            def full_src(v0):
                return tt_hbm.at[w, :, pl.ds(pl.multiple_of(v0, 128), WLEN)]

            def coarse(cb, kp):
                c0 = cb * CLEN
                cnt_c = scan(None, None, B, clist_v, cblist_v,
                             c0, c0 + CLEN, True)
                def win_body(wi, kp_):
                    win = cb * 8 + wi
                    v0 = win * WLEN

                    def pf():
                        @pl.when(win + 1 <= NWIN_FULL - 1)
                        def _():
                            nxt = win + 1
                            pltpu.async_copy(full_src(nxt * WLEN),
                                             blk.at[nxt & 1],
                                             wsem.at[nxt & 1])
                    return process_window(v0, WLEN, v0 + WLEN, full_src(v0),
                                          win & 1, cnt_c, kp_, prefetch=pf)
                nw_full = lax.min(jnp.int32(8), NWIN_FULL - cb * 8)
                kp = lax.fori_loop(0, nw_full, win_body, kp)
                return kp, cnt_c

            pltpu.async_copy(full_src(0), blk.at[0], wsem.at[0])
            kp = lax.fori_loop(
                0, NCOARSE - 1, lambda cb, kp_: coarse(cb, kp_)[0],
                jnp.int32(0))
            v0t = NWIN_FULL * WLEN
            tail_src = tt_hbm.at[w, :, pl.ds(v0t, TAIL)]
            pltpu.async_copy(tail_src, blk.at[1, :, pl.ds(0, TAIL)],
                             wsem.at[1])
            kp, cnt_c = coarse(NCOARSE - 1, kp)

            def pf_slab():
                pltpu.async_copy(slab_hbm.at[w], blk.at[0, :, pl.ds(0, 128)],
                                 wsem.at[0])
            kp = process_window(v0t, TAIL, v0t + TAIL, tail_src, 1,
                                cnt_c, kp, prefetch=pf_slab)
            process_window(VA, 128, V, slab_hbm.at[w], 0, cnt_c, kp)
            for qq in (NFLUSH - 2, NFLUSH - 1):
                qp = qq & 1
                pltpu.make_async_copy(
                    pack.at[qp],
                    packed_hbm.at[pl.ds(w * (B // 4) + qq * 32, 32)],
                    fsem.at[qp]).wait()
                pltpu.make_async_copy(
                    bidx.at[qp], bidx_hbm.at[w, pl.ds(qq, 1)],
                    fsem.at[qp]).wait()

    return k1


@functools.lru_cache(maxsize=None)
def _make_k2():
    mesh = plsc.VectorSubcoreMesh(core_axis_name="c", subcore_axis_name="s")

    @functools.partial(
        pl.kernel,
        mesh=mesh,
        out_type=jax.ShapeDtypeStruct((B * F, D), jnp.float32),
        scratch_types=[
            pltpu.VMEM((NFLUSH, 128), jnp.int32),
            pltpu.VMEM((B // 2, D), jnp.float32),
            pltpu.SemaphoreType.DMA,
            pltpu.SemaphoreType.DMA,
        ],
        compiler_params=pltpu.CompilerParams(use_tc_tiling_on_sc=False),
    )
    def k2(packed_hbm, bidx_hbm, out_hbm, bidx_v, rows_v, lsem, ssem):
        w = lax.axis_index("s") * NC + lax.axis_index("c")

        @pl.when(w < F)
        def _body():
            pltpu.sync_copy(bidx_hbm.at[w], bidx_v)
            for half in range(2):
                pltpu.sync_copy(
                    packed_hbm.at[pl.ds(w * B + half * (B // 2), B // 2)],
                    rows_v)
                scs = []
                for j in range(NFLUSH // 2):
                    scs.append(pltpu.async_copy(
                        rows_v.at[pl.ds(j * 128, 128)],
                        out_hbm.at[bidx_v.at[half * (NFLUSH // 2) + j]],
                        ssem))
                for s in scs:
                    s.wait()

    return k2


BB = 512


def _mlp_body(emb_ref, num_ref, w1a_ref, w1b_ref, b1_ref, w2_ref, b2_ref,
              w3_ref, b3_ref, w4_ref, b4_ref, out_ref):
    h = jnp.dot(emb_ref[...], w1a_ref[...], preferred_element_type=jnp.float32)
    h += jnp.dot(num_ref[...], w1b_ref[...], preferred_element_type=jnp.float32)
    h = jnp.maximum(h + b1_ref[...], 0.0)
    h = jnp.dot(h, w2_ref[...], preferred_element_type=jnp.float32)
    h = jnp.maximum(h + b2_ref[...], 0.0)
    h = jnp.dot(h, w3_ref[...], preferred_element_type=jnp.float32)
    h = jnp.maximum(h + b3_ref[...], 0.0)
    out_ref[...] = (
        jnp.dot(h, w4_ref[...], preferred_element_type=jnp.float32) + b4_ref[...]
    )


def _mlp(emb, num, w1a, w1b, b1, w2, b2, w3, b3, w4, b4):
    grid = B // BB
    full = lambda i: (0, 0)
    return pl.pallas_call(
        _mlp_body,
        grid=(grid,),
        in_specs=[
            pl.BlockSpec((BB, F * D), lambda i: (i, 0)),
            pl.BlockSpec((BB, NUM), lambda i: (i, 0)),
            pl.BlockSpec((F * D, 512), full),
            pl.BlockSpec((NUM, 512), full),
            pl.BlockSpec((1, 512), full),
            pl.BlockSpec((512, 256), full),
            pl.BlockSpec((1, 256), full),
            pl.BlockSpec((256, 128), full),
            pl.BlockSpec((1, 128), full),
            pl.BlockSpec((128, 1), full),
            pl.BlockSpec((1, 1), full),
        ],
        out_specs=pl.BlockSpec((BB, 1), lambda i: (i, 0)),
        out_shape=jax.ShapeDtypeStruct((B, 1), jnp.float32),
    )(emb, num, w1a, w1b, b1, w2, b2, w3, b3, w4, b4)


def kernel(numerical_features, cat_features, tables, W1, b1, W2, b2, W3, b3,
           W4, b4):
    tt = tables.transpose(0, 2, 1)
    slab = jnp.pad(tt[:, :, VA:], ((0, 0), (0, 0), (0, 128 - (V - VA))))
    cat3 = cat_features.reshape(F, 1, B)
    packed, bidx = _make_k1()(tt, slab, cat3)
    flat = packed.reshape(F * B, D)
    emb2 = _make_k2()(flat, bidx).reshape(B, F * D)
    return _mlp(
        emb2,
        numerical_features,
        W1[: F * D],
        W1[F * D :],
        b1.reshape(1, -1),
        W2,
        b2.reshape(1, -1),
        W3,
        b3.reshape(1, -1),
        W4,
        b4.reshape(1, 1),
    )


# speedup vs baseline: 4.6220x; 1.0326x over previous
import functools

import jax
import jax.numpy as jnp
from jax import lax
from jax.experimental import pallas as pl
from jax.experimental.pallas import tpu as pltpu
from jax.experimental.pallas import tpu_sc as plsc

B = 4096
F = 26
V = 100000
D = 32
NUM = 13

NC = 2
NS = 16
NW = NC * NS

WLEN = 1536
NWIN_FULL = V // WLEN
TAIL = 128
VA = NWIN_FULL * WLEN + TAIL
CLEN = 8 * WLEN
NCOARSE = (V + CLEN - 1) // CLEN

NFLUSH = B // 128


def _iota16():
    return lax.iota(jnp.int32, 16)


@functools.lru_cache(maxsize=None)
def _make_k1():
    mesh = plsc.VectorSubcoreMesh(core_axis_name="c", subcore_axis_name="s")

    @functools.partial(
        pl.kernel,
        mesh=mesh,
        out_type=(
            jax.ShapeDtypeStruct((F * B // 4, 128), jnp.float32),
            jax.ShapeDtypeStruct((F, NFLUSH, 128), jnp.int32),
        ),
        scratch_types=[
            pltpu.VMEM((D, B // D), jnp.int32),
            pltpu.VMEM((B + 32,), jnp.int32),
            pltpu.VMEM((B + 32,), jnp.int32),
            pltpu.VMEM((B + 32,), jnp.int32),
            pltpu.VMEM((B + 32,), jnp.int32),
            pltpu.VMEM((2, D, WLEN), jnp.float32),
            pltpu.VMEM((2, D, 128), jnp.float32),
            pltpu.VMEM((2, 1, 128), jnp.int32),
            pltpu.SemaphoreType.DMA((2,)),
            pltpu.SemaphoreType.DMA((2,)),
        ],
        compiler_params=pltpu.CompilerParams(
            use_tc_tiling_on_sc=True, needs_layout_passes=False
        ),
    )
    def k1(tt_hbm, slab_hbm, cat_hbm, packed_hbm, bidx_hbm,
           idx_v, clist_v, cblist_v, vlist_v, blist_v, blk, pack, bidx,
           wsem, fsem):
        w = lax.axis_index("s") * NC + lax.axis_index("c")

        @pl.when(w < F)
        def _body():
            pltpu.sync_copy(cat_hbm.at[w], idx_v)

            def scan(src_ref, bsrc_ref, limit, dst_ref, bdst_ref, lo, hi, first):
                def chunk(c, cnt):
                    lanes = c * 16 + _iota16()
                    if first:
                        v = idx_v[c >> 3, pl.ds((c & 7) * 16, 16)]
                        b = lanes
                    else:
                        v = src_ref[pl.ds(c * 16, 16)]
                        b = bsrc_ref[pl.ds(c * 16, 16)]
                    m = (v >= lo) & (v < hi) & (lanes < limit)
                    plsc.store_compressed(dst_ref.at[pl.ds(cnt, 16)], v, mask=m)
                    plsc.store_compressed(bdst_ref.at[pl.ds(cnt, 16)], b, mask=m)
                    npos = plsc.all_reduce_population_count(m)
                    return cnt + lax.reduce_max(npos, (0,))
                n16 = (limit + 15) >> 4 if not isinstance(limit, int) else (limit + 15) // 16
                return lax.fori_loop(0, n16, chunk, jnp.int32(0))

            def process_window(v0, wl, hi, src, par, cnt_c, kp0,
                               prefetch=None):
                if prefetch is not None:
                    prefetch()
                wcnt = scan(clist_v, cblist_v, cnt_c,
                            vlist_v, blist_v, v0, hi, False)
                pltpu.make_async_copy(
                    src, blk.at[par, :, pl.ds(0, wl)], wsem.at[par]).wait()
                parv = jnp.full((16,), par, jnp.int32)

                def match16(g, kp):
                    base = g * 16
                    nv = lax.min(jnp.int32(16), wcnt - base)
                    kp_new = kp + nv
                    nb = (kp_new - 1) >> 7

                    @pl.when(((nb > (kp >> 7)) | ((kp & 127) == 0))
                             & (nb >= 2))
                    def _drain():
                        q_old = nb - 2
                        qp = q_old & 1
                        pltpu.make_async_copy(
                            pack.at[qp],
                            packed_hbm.at[pl.ds(w * (B // 4) + q_old * 32, 32)],
                            fsem.at[qp]).wait()
                        pltpu.make_async_copy(
                            bidx.at[qp], bidx_hbm.at[w, pl.ds(q_old, 1)],
                            fsem.at[qp]).wait()

                    vv = vlist_v[pl.ds(base, 16)]
                    bb = blist_v[pl.ds(base, 16)]
                    valid = _iota16() < nv
                    cols = vv - v0
                    pos = kp + _iota16()
                    ppv = (pos >> 7) & 1
                    bp32 = (pos & 127) * 32
                    for d in range(D):
                        vals = plsc.load_gather(
                            blk, [parv, jnp.full((16,), d, jnp.int32), cols],
                            mask=valid)
                        fl = bp32 + d
                        plsc.store_scatter(
                            pack, [ppv, fl >> 7, fl & 127], vals, mask=valid)
                    plsc.store_scatter(
                        bidx,
                        [ppv, jnp.zeros((16,), jnp.int32), pos & 127],
                        bb * F + w, mask=valid)

                    @pl.when((kp_new >> 7) > (kp >> 7))
                    def _flush():
                        q = (kp_new >> 7) - 1
                        qp = q & 1
                        pltpu.async_copy(
                            pack.at[qp],
                            packed_hbm.at[pl.ds(w * (B // 4) + q * 32, 32)],
                            fsem.at[qp])
                        pltpu.async_copy(
                            bidx.at[qp], bidx_hbm.at[w, pl.ds(q, 1)],
                            fsem.at[qp])
                    return kp_new
                ngrp = (wcnt + 15) >> 4
                return lax.fori_loop(0, ngrp, match16, kp0)

            def full_src(v0):
                return tt_hbm.at[w, :, pl.ds(pl.multiple_of(v0, 128), WLEN)]

            def coarse(cb, kp):
                c0 = cb * CLEN
                cnt_c = scan(None, None, B, clist_v, cblist_v,
                             c0, c0 + CLEN, True)
                def win_body(wi, kp_):
                    win = cb * 8 + wi
                    v0 = win * WLEN

                    def pf():
                        @pl.when(win + 1 <= NWIN_FULL - 1)
                        def _():
                            nxt = win + 1
                            pltpu.async_copy(full_src(nxt * WLEN),
                                             blk.at[nxt & 1],
                                             wsem.at[nxt & 1])
                    return process_window(v0, WLEN, v0 + WLEN, full_src(v0),
                                          win & 1, cnt_c, kp_, prefetch=pf)
                nw_full = lax.min(jnp.int32(8), NWIN_FULL - cb * 8)
                kp = lax.fori_loop(0, nw_full, win_body, kp)
                return kp, cnt_c

            pltpu.async_copy(full_src(0), blk.at[0], wsem.at[0])
            kp = lax.fori_loop(
                0, NCOARSE - 1, lambda cb, kp_: coarse(cb, kp_)[0],
                jnp.int32(0))
            v0t = NWIN_FULL * WLEN
            tail_src = tt_hbm.at[w, :, pl.ds(v0t, TAIL)]
            pltpu.async_copy(tail_src, blk.at[1, :, pl.ds(0, TAIL)],
                             wsem.at[1])
            kp, cnt_c = coarse(NCOARSE - 1, kp)

            def pf_slab():
                pltpu.async_copy(slab_hbm.at[w], blk.at[0, :, pl.ds(0, 128)],
                                 wsem.at[0])
            kp = process_window(v0t, TAIL, v0t + TAIL, tail_src, 1,
                                cnt_c, kp, prefetch=pf_slab)
            process_window(VA, 128, V, slab_hbm.at[w], 0, cnt_c, kp)
            for qq in (NFLUSH - 2, NFLUSH - 1):
                qp = qq & 1
                pltpu.make_async_copy(
                    pack.at[qp],
                    packed_hbm.at[pl.ds(w * (B // 4) + qq * 32, 32)],
                    fsem.at[qp]).wait()
                pltpu.make_async_copy(
                    bidx.at[qp], bidx_hbm.at[w, pl.ds(qq, 1)],
                    fsem.at[qp]).wait()

    return k1


@functools.lru_cache(maxsize=None)
def _make_k2():
    mesh = plsc.VectorSubcoreMesh(core_axis_name="c", subcore_axis_name="s")

    @functools.partial(
        pl.kernel,
        mesh=mesh,
        out_type=jax.ShapeDtypeStruct((B * F, D), jnp.float32),
        scratch_types=[
            pltpu.VMEM((NFLUSH, 128), jnp.int32),
            pltpu.VMEM((B // 2, D), jnp.float32),
            pltpu.SemaphoreType.DMA,
            pltpu.SemaphoreType.DMA,
        ],
        compiler_params=pltpu.CompilerParams(use_tc_tiling_on_sc=False),
    )
    def k2(packed_hbm, bidx_hbm, out_hbm, bidx_v, rows_v, lsem, ssem):
        w = lax.axis_index("s") * NC + lax.axis_index("c")

        @pl.when(w < F)
        def _body():
            pltpu.sync_copy(bidx_hbm.at[w], bidx_v)
            for half in range(2):
                pltpu.sync_copy(
                    packed_hbm.at[pl.ds(w * B + half * (B // 2), B // 2)],
                    rows_v)
                scs = []
                for j in range(NFLUSH // 2):
                    scs.append(pltpu.async_copy(
                        rows_v.at[pl.ds(j * 128, 128)],
                        out_hbm.at[bidx_v.at[half * (NFLUSH // 2) + j]],
                        ssem))
                for s in scs:
                    s.wait()

    return k2


BB = 512


def _mlp_body(emb_ref, num_ref, w1a_ref, w1b_ref, b1_ref, w2_ref, b2_ref,
              w3_ref, b3_ref, w4_ref, b4_ref, out_ref):
    h = jnp.dot(emb_ref[...], w1a_ref[...], preferred_element_type=jnp.float32)
    h += jnp.dot(num_ref[...], w1b_ref[...], preferred_element_type=jnp.float32)
    h = jnp.maximum(h + b1_ref[...], 0.0)
    h = jnp.dot(h, w2_ref[...], preferred_element_type=jnp.float32)
    h = jnp.maximum(h + b2_ref[...], 0.0)
    h = jnp.dot(h, w3_ref[...], preferred_element_type=jnp.float32)
    h = jnp.maximum(h + b3_ref[...], 0.0)
    out_ref[...] = (
        jnp.dot(h, w4_ref[...], preferred_element_type=jnp.float32) + b4_ref[...]
    )


def _mlp(emb, num, w1a, w1b, b1, w2, b2, w3, b3, w4, b4):
    grid = B // BB
    full = lambda i: (0, 0)
    return pl.pallas_call(
        _mlp_body,
        grid=(grid,),
        in_specs=[
            pl.BlockSpec((BB, F * D), lambda i: (i, 0)),
            pl.BlockSpec((BB, NUM), lambda i: (i, 0)),
            pl.BlockSpec((F * D, 512), full),
            pl.BlockSpec((NUM, 512), full),
            pl.BlockSpec((1, 512), full),
            pl.BlockSpec((512, 256), full),
            pl.BlockSpec((1, 256), full),
            pl.BlockSpec((256, 128), full),
            pl.BlockSpec((1, 128), full),
            pl.BlockSpec((128, 1), full),
            pl.BlockSpec((1, 1), full),
        ],
        out_specs=pl.BlockSpec((BB, 1), lambda i: (i, 0)),
        out_shape=jax.ShapeDtypeStruct((B, 1), jnp.float32),
    )(emb, num, w1a, w1b, b1, w2, b2, w3, b3, w4, b4)


def kernel(numerical_features, cat_features, tables, W1, b1, W2, b2, W3, b3,
           W4, b4):
    tt = tables.transpose(0, 2, 1)
    slab = jnp.pad(tt[:, :, VA:], ((0, 0), (0, 0), (0, 128 - (V - VA))))
    cat3 = cat_features.reshape(F, D, B // D)
    packed, bidx = _make_k1()(tt, slab, cat3)
    flat = packed.reshape(F * B, D)
    emb2 = _make_k2()(flat, bidx).reshape(B, F * D)
    return _mlp(
        emb2,
        numerical_features,
        W1[: F * D],
        W1[F * D :],
        b1.reshape(1, -1),
        W2,
        b2.reshape(1, -1),
        W3,
        b3.reshape(1, -1),
        W4,
        b4.reshape(1, 1),
    )


# speedup vs baseline: 4.7602x; 1.0299x over previous
import functools

import jax
import jax.numpy as jnp
from jax import lax
from jax.experimental import pallas as pl
from jax.experimental.pallas import tpu as pltpu
from jax.experimental.pallas import tpu_sc as plsc

B = 4096
F = 26
V = 100000
D = 32
NUM = 13

NC = 2
NS = 16
NW = NC * NS

WLEN = 1024
NWIN_FULL = V // WLEN
TAIL = 640
VA = NWIN_FULL * WLEN + TAIL
CLEN = 8 * WLEN
NCOARSE = (V + CLEN - 1) // CLEN
NBUF = 3

NFLUSH = B // 128


def _iota16():
    return lax.iota(jnp.int32, 16)


@functools.lru_cache(maxsize=None)
def _make_k1():
    mesh = plsc.VectorSubcoreMesh(core_axis_name="c", subcore_axis_name="s")

    @functools.partial(
        pl.kernel,
        mesh=mesh,
        out_type=(
            jax.ShapeDtypeStruct((F * B // 4, 128), jnp.float32),
            jax.ShapeDtypeStruct((F, NFLUSH, 128), jnp.int32),
        ),
        scratch_types=[
            pltpu.VMEM((D, B // D), jnp.int32),
            pltpu.VMEM((B + 32,), jnp.int32),
            pltpu.VMEM((B + 32,), jnp.int32),
            pltpu.VMEM((B + 32,), jnp.int32),
            pltpu.VMEM((B + 32,), jnp.int32),
            pltpu.VMEM((NBUF, D, WLEN), jnp.float32),
            pltpu.VMEM((2, D, 128), jnp.float32),
            pltpu.VMEM((2, 1, 128), jnp.int32),
            pltpu.SemaphoreType.DMA((NBUF,)),
            pltpu.SemaphoreType.DMA((2,)),
        ],
        compiler_params=pltpu.CompilerParams(
            use_tc_tiling_on_sc=True, needs_layout_passes=False
        ),
    )
    def k1(tt_hbm, slab_hbm, cat_hbm, packed_hbm, bidx_hbm,
           idx_v, clist_v, cblist_v, vlist_v, blist_v, blk, pack, bidx,
           wsem, fsem):
        w = lax.axis_index("s") * NC + lax.axis_index("c")

        @pl.when(w < F)
        def _body():
            pltpu.sync_copy(cat_hbm.at[w], idx_v)

            def scan(src_ref, bsrc_ref, limit, dst_ref, bdst_ref, lo, hi, first):
                def chunk(c, cnt):
                    lanes = c * 16 + _iota16()
                    if first:
                        v = idx_v[c >> 3, pl.ds((c & 7) * 16, 16)]
                        b = lanes
                    else:
                        v = src_ref[pl.ds(c * 16, 16)]
                        b = bsrc_ref[pl.ds(c * 16, 16)]
                    m = (v >= lo) & (v < hi) & (lanes < limit)
                    plsc.store_compressed(dst_ref.at[pl.ds(cnt, 16)], v, mask=m)
                    plsc.store_compressed(bdst_ref.at[pl.ds(cnt, 16)], b, mask=m)
                    npos = plsc.all_reduce_population_count(m)
                    return cnt + lax.reduce_max(npos, (0,))
                n16 = (limit + 15) >> 4 if not isinstance(limit, int) else (limit + 15) // 16
                return lax.fori_loop(0, n16, chunk, jnp.int32(0))

            def process_window(v0, wl, hi, src, par, cnt_c, kp0,
                               prefetch=None):
                if prefetch is not None:
                    prefetch()
                wcnt = scan(clist_v, cblist_v, cnt_c,
                            vlist_v, blist_v, v0, hi, False)
                pltpu.make_async_copy(
                    src, blk.at[par, :, pl.ds(0, wl)], wsem.at[par]).wait()
                parv = jnp.full((16,), par, jnp.int32)

                def match16(g, kp):
                    base = g * 16
                    nv = lax.min(jnp.int32(16), wcnt - base)
                    kp_new = kp + nv
                    nb = (kp_new - 1) >> 7

                    @pl.when(((nb > (kp >> 7)) | ((kp & 127) == 0))
                             & (nb >= 2))
                    def _drain():
                        q_old = nb - 2
                        qp = q_old & 1
                        pltpu.make_async_copy(
                            pack.at[qp],
                            packed_hbm.at[pl.ds(w * (B // 4) + q_old * 32, 32)],
                            fsem.at[qp]).wait()
                        pltpu.make_async_copy(
                            bidx.at[qp], bidx_hbm.at[w, pl.ds(q_old, 1)],
                            fsem.at[qp]).wait()

                    vv = vlist_v[pl.ds(base, 16)]
                    bb = blist_v[pl.ds(base, 16)]
                    valid = _iota16() < nv
                    cols = vv - v0
                    pos = kp + _iota16()
                    ppv = (pos >> 7) & 1
                    bp32 = (pos & 127) * 32
                    for d in range(D):
                        vals = plsc.load_gather(
                            blk, [parv, jnp.full((16,), d, jnp.int32), cols],
                            mask=valid)
                        fl = bp32 + d
                        plsc.store_scatter(
                            pack, [ppv, fl >> 7, fl & 127], vals, mask=valid)
                    plsc.store_scatter(
                        bidx,
                        [ppv, jnp.zeros((16,), jnp.int32), pos & 127],
                        bb * F + w, mask=valid)

                    @pl.when((kp_new >> 7) > (kp >> 7))
                    def _flush():
                        q = (kp_new >> 7) - 1
                        qp = q & 1
                        pltpu.async_copy(
                            pack.at[qp],
                            packed_hbm.at[pl.ds(w * (B // 4) + q * 32, 32)],
                            fsem.at[qp])
                        pltpu.async_copy(
                            bidx.at[qp], bidx_hbm.at[w, pl.ds(q, 1)],
                            fsem.at[qp])
                    return kp_new
                ngrp = (wcnt + 15) >> 4
                return lax.fori_loop(0, ngrp, match16, kp0)

            def full_src(v0):
                return tt_hbm.at[w, :, pl.ds(pl.multiple_of(v0, 128), WLEN)]

            def coarse(cb, kp):
                c0 = cb * CLEN
                cnt_c = scan(None, None, B, clist_v, cblist_v,
                             c0, c0 + CLEN, True)
                def win_body(wi, kp_):
                    win = cb * 8 + wi
                    v0 = win * WLEN

                    def pf():
                        @pl.when(win + 2 <= NWIN_FULL - 1)
                        def _():
                            nxt = win + 2
                            pltpu.async_copy(full_src(nxt * WLEN),
                                             blk.at[nxt % NBUF],
                                             wsem.at[nxt % NBUF])
                    return process_window(v0, WLEN, v0 + WLEN, full_src(v0),
                                          win % NBUF, cnt_c, kp_, prefetch=pf)
                nw_full = lax.min(jnp.int32(8), NWIN_FULL - cb * 8)
                kp = lax.fori_loop(0, nw_full, win_body, kp)
                return kp, cnt_c

            pltpu.async_copy(full_src(0), blk.at[0], wsem.at[0])
            pltpu.async_copy(full_src(WLEN), blk.at[1], wsem.at[1])
            kp = lax.fori_loop(
                0, NCOARSE - 1, lambda cb, kp_: coarse(cb, kp_)[0],
                jnp.int32(0))
            v0t = NWIN_FULL * WLEN
            tail_src = tt_hbm.at[w, :, pl.ds(v0t, TAIL)]
            pltpu.async_copy(tail_src, blk.at[1, :, pl.ds(0, TAIL)],
                             wsem.at[1])
            pltpu.async_copy(slab_hbm.at[w], blk.at[2, :, pl.ds(0, 128)],
                             wsem.at[2])
            kp, cnt_c = coarse(NCOARSE - 1, kp)
            kp = process_window(v0t, TAIL, v0t + TAIL, tail_src, 1,
                                cnt_c, kp)
            process_window(VA, 128, V, slab_hbm.at[w], 2, cnt_c, kp)
            for qq in (NFLUSH - 2, NFLUSH - 1):
                qp = qq & 1
                pltpu.make_async_copy(
                    pack.at[qp],
                    packed_hbm.at[pl.ds(w * (B // 4) + qq * 32, 32)],
                    fsem.at[qp]).wait()
                pltpu.make_async_copy(
                    bidx.at[qp], bidx_hbm.at[w, pl.ds(qq, 1)],
                    fsem.at[qp]).wait()

    return k1


@functools.lru_cache(maxsize=None)
def _make_k2():
    mesh = plsc.VectorSubcoreMesh(core_axis_name="c", subcore_axis_name="s")

    @functools.partial(
        pl.kernel,
        mesh=mesh,
        out_type=jax.ShapeDtypeStruct((B * F, D), jnp.float32),
        scratch_types=[
            pltpu.VMEM((NFLUSH, 128), jnp.int32),
            pltpu.VMEM((B // 2, D), jnp.float32),
            pltpu.SemaphoreType.DMA,
            pltpu.SemaphoreType.DMA,
        ],
        compiler_params=pltpu.CompilerParams(use_tc_tiling_on_sc=False),
    )
    def k2(packed_hbm, bidx_hbm, out_hbm, bidx_v, rows_v, lsem, ssem):
        w = lax.axis_index("s") * NC + lax.axis_index("c")

        @pl.when(w < F)
        def _body():
            pltpu.sync_copy(bidx_hbm.at[w], bidx_v)
            for half in range(2):
                pltpu.sync_copy(
                    packed_hbm.at[pl.ds(w * B + half * (B // 2), B // 2)],
                    rows_v)
                scs = []
                for j in range(NFLUSH // 2):
                    scs.append(pltpu.async_copy(
                        rows_v.at[pl.ds(j * 128, 128)],
                        out_hbm.at[bidx_v.at[half * (NFLUSH // 2) + j]],
                        ssem))
                for s in scs:
                    s.wait()

    return k2


BB = 512


def _mlp_body(emb_ref, num_ref, w1a_ref, w1b_ref, b1_ref, w2_ref, b2_ref,
              w3_ref, b3_ref, w4_ref, b4_ref, out_ref):
    h = jnp.dot(emb_ref[...], w1a_ref[...], preferred_element_type=jnp.float32)
    h += jnp.dot(num_ref[...], w1b_ref[...], preferred_element_type=jnp.float32)
    h = jnp.maximum(h + b1_ref[...], 0.0)
    h = jnp.dot(h, w2_ref[...], preferred_element_type=jnp.float32)
    h = jnp.maximum(h + b2_ref[...], 0.0)
    h = jnp.dot(h, w3_ref[...], preferred_element_type=jnp.float32)
    h = jnp.maximum(h + b3_ref[...], 0.0)
    out_ref[...] = (
        jnp.dot(h, w4_ref[...], preferred_element_type=jnp.float32) + b4_ref[...]
    )


def _mlp(emb, num, w1a, w1b, b1, w2, b2, w3, b3, w4, b4):
    grid = B // BB
    full = lambda i: (0, 0)
    return pl.pallas_call(
        _mlp_body,
        grid=(grid,),
        in_specs=[
            pl.BlockSpec((BB, F * D), lambda i: (i, 0)),
            pl.BlockSpec((BB, NUM), lambda i: (i, 0)),
            pl.BlockSpec((F * D, 512), full),
            pl.BlockSpec((NUM, 512), full),
            pl.BlockSpec((1, 512), full),
            pl.BlockSpec((512, 256), full),
            pl.BlockSpec((1, 256), full),
            pl.BlockSpec((256, 128), full),
            pl.BlockSpec((1, 128), full),
            pl.BlockSpec((128, 1), full),
            pl.BlockSpec((1, 1), full),
        ],
        out_specs=pl.BlockSpec((BB, 1), lambda i: (i, 0)),
        out_shape=jax.ShapeDtypeStruct((B, 1), jnp.float32),
    )(emb, num, w1a, w1b, b1, w2, b2, w3, b3, w4, b4)


def kernel(numerical_features, cat_features, tables, W1, b1, W2, b2, W3, b3,
           W4, b4):
    tt = tables.transpose(0, 2, 1)
    slab = jnp.pad(tt[:, :, VA:], ((0, 0), (0, 0), (0, 128 - (V - VA))))
    cat3 = cat_features.reshape(F, D, B // D)
    packed, bidx = _make_k1()(tt, slab, cat3)
    flat = packed.reshape(F * B, D)
    emb2 = _make_k2()(flat, bidx).reshape(B, F * D)
    return _mlp(
        emb2,
        numerical_features,
        W1[: F * D],
        W1[F * D :],
        b1.reshape(1, -1),
        W2,
        b2.reshape(1, -1),
        W3,
        b3.reshape(1, -1),
        W4,
        b4.reshape(1, 1),
    )
